# Initial kernel scaffold; baseline (speedup 1.0000x reference)
import jax
import jax.numpy as jnp
from jax.experimental import pallas as pl


def kernel(z_latents, embeddings):
    raise NotImplementedError("write your pallas kernel here")


# speedup vs baseline: 3.7058x; 3.7058x over previous
import functools

import jax
import jax.numpy as jnp
from jax import lax
from jax.experimental import pallas as pl
from jax.experimental.pallas import tpu as pltpu
from jax.experimental.pallas import tpu_sc as plsc

_EMBEDDING_DIM = 32
_NUM_EMBEDDINGS = 8192
_COMMITMENT_COST = 0.25

_ROWS = 8192
_ROW_BLOCK = 256
_NUM_ROW_BLOCKS = _ROWS // _ROW_BLOCK

_SC_CORES = 2
_SC_SUBCORES = 16
_SC_WORKERS = _SC_CORES * _SC_SUBCORES
_IDX_CHUNK = 128
_CHUNKS_PER_WORKER = _ROWS // (_SC_WORKERS * _IDX_CHUNK)


def _argmin_body(x_ref, e_ref, c_ref, s_ref, o_ref):
    x = x_ref[...]
    e = e_ref[...]
    dot = lax.dot_general(
        x, e, (((1,), (0,)), ((), ())),
        preferred_element_type=jnp.float32,
    )
    d = (c_ref[...] - 2.0 * dot) + s_ref[...]
    m = jnp.min(d, axis=1, keepdims=True)
    ji = lax.broadcasted_iota(jnp.int32, d.shape, 1)
    idx = jnp.min(jnp.where(d == m, ji, jnp.int32(2**31 - 1)), axis=1)
    o_ref[0, 0, :] = idx


def _argmin_indices(flat, embeddings, inputs_sqr, emb_sqr):
    out = pl.pallas_call(
        _argmin_body,
        grid=(_NUM_ROW_BLOCKS,),
        in_specs=[
            pl.BlockSpec((_ROW_BLOCK, _EMBEDDING_DIM), lambda i: (i, 0)),
            pl.BlockSpec((_EMBEDDING_DIM, _NUM_EMBEDDINGS), lambda i: (0, 0)),
            pl.BlockSpec((_ROW_BLOCK, 1), lambda i: (i, 0)),
            pl.BlockSpec((1, _NUM_EMBEDDINGS), lambda i: (0, 0)),
        ],
        out_specs=pl.BlockSpec((1, 1, _ROW_BLOCK), lambda i: (i, 0, 0)),
        out_shape=jax.ShapeDtypeStruct(
            (_NUM_ROW_BLOCKS, 1, _ROW_BLOCK), jnp.int32),
    )(flat, embeddings, inputs_sqr, emb_sqr)
    return out.reshape(_ROWS // _IDX_CHUNK, _IDX_CHUNK)


def _sc_gather_body(idx_hbm, table_hbm, out_hbm, idx_v, rows_v, sem):
    wid = lax.axis_index("s") * _SC_CORES + lax.axis_index("c")
    base = wid * _CHUNKS_PER_WORKER
    pltpu.sync_copy(idx_hbm.at[pl.ds(base, _CHUNKS_PER_WORKER)], idx_v)
    copies = []
    for j in range(_CHUNKS_PER_WORKER):
        copies.append(
            pltpu.async_copy(table_hbm.at[idx_v.at[j]], rows_v.at[j], sem))
    for c in copies:
        c.wait()
    pltpu.sync_copy(rows_v, out_hbm.at[pl.ds(base, _CHUNKS_PER_WORKER)])


@functools.cache
def _sc_gather():
    return pl.kernel(
        _sc_gather_body,
        mesh=plsc.VectorSubcoreMesh(core_axis_name="c", subcore_axis_name="s"),
        out_type=jax.ShapeDtypeStruct(
            (_ROWS // _IDX_CHUNK, _IDX_CHUNK, _EMBEDDING_DIM), jnp.float32),
        scratch_types=[
            pltpu.VMEM((_CHUNKS_PER_WORKER, _IDX_CHUNK), jnp.int32),
            pltpu.VMEM((_CHUNKS_PER_WORKER, _IDX_CHUNK, _EMBEDDING_DIM),
                       jnp.float32),
            pltpu.SemaphoreType.DMA,
        ],
        compiler_params=pltpu.CompilerParams(use_tc_tiling_on_sc=False),
    )


def _epilogue_body(z_ref, q_ref, out_ref, loss_ref):
    z = z_ref[...]
    q = q_ref[...]
    diff = q - z
    out_ref[...] = z + diff
    m = jnp.sum(diff * diff) * (1.0 / float(_ROWS * _EMBEDDING_DIM))
    loss_ref[0, 0] = m + _COMMITMENT_COST * m


def _epilogue(flat, q):
    return pl.pallas_call(
        _epilogue_body,
        out_shape=(
            jax.ShapeDtypeStruct((_ROWS, _EMBEDDING_DIM), jnp.float32),
            jax.ShapeDtypeStruct((1, 1), jnp.float32),
        ),
        out_specs=(
            pl.BlockSpec(memory_space=pltpu.VMEM),
            pl.BlockSpec(memory_space=pltpu.SMEM),
        ),
    )(flat, q)


def kernel(z_latents, embeddings):
    z_shape = z_latents.shape
    flat = z_latents.reshape(-1, _EMBEDDING_DIM)
    inputs_sqr = jnp.sum(flat**2, axis=1, keepdims=True)
    emb_sqr = jnp.sum(embeddings**2, axis=0, keepdims=True)
    idx = _argmin_indices(flat, embeddings, inputs_sqr, emb_sqr)
    table = embeddings.T
    q = _sc_gather()(idx, table).reshape(_ROWS, _EMBEDDING_DIM)
    out, loss = _epilogue(flat, q)
    return out.reshape(z_shape), loss[0, 0]


# speedup vs baseline: 3.8099x; 1.0281x over previous
import functools

import jax
import jax.numpy as jnp
from jax import lax
from jax.experimental import pallas as pl
from jax.experimental.pallas import tpu as pltpu
from jax.experimental.pallas import tpu_sc as plsc

_EMBEDDING_DIM = 32
_NUM_EMBEDDINGS = 8192
_COMMITMENT_COST = 0.25

_ROWS = 8192
_ROW_BLOCK = 512
_NUM_ROW_BLOCKS = _ROWS // _ROW_BLOCK

_SC_CORES = 2
_SC_SUBCORES = 16
_SC_WORKERS = _SC_CORES * _SC_SUBCORES
_IDX_CHUNK = 128
_CHUNKS_PER_WORKER = _ROWS // (_SC_WORKERS * _IDX_CHUNK)


def _argmin_body(x_ref, e_ref, c_ref, s_ref, o_ref):
    x = x_ref[...]
    e = e_ref[...]
    dot = lax.dot_general(
        x, e, (((1,), (0,)), ((), ())),
        preferred_element_type=jnp.float32,
    )
    d = (c_ref[...] - 2.0 * dot) + s_ref[...]
    m = jnp.min(d, axis=1, keepdims=True)
    ji = lax.broadcasted_iota(jnp.int32, d.shape, 1)
    idx = jnp.min(jnp.where(d == m, ji, jnp.int32(2**31 - 1)), axis=1)
    o_ref[0, 0, :] = idx


def _argmin_indices(flat, embeddings, inputs_sqr, emb_sqr):
    out = pl.pallas_call(
        _argmin_body,
        grid=(_NUM_ROW_BLOCKS,),
        in_specs=[
            pl.BlockSpec((_ROW_BLOCK, _EMBEDDING_DIM), lambda i: (i, 0)),
            pl.BlockSpec((_EMBEDDING_DIM, _NUM_EMBEDDINGS), lambda i: (0, 0)),
            pl.BlockSpec((_ROW_BLOCK, 1), lambda i: (i, 0)),
            pl.BlockSpec((1, _NUM_EMBEDDINGS), lambda i: (0, 0)),
        ],
        out_specs=pl.BlockSpec((1, 1, _ROW_BLOCK), lambda i: (i, 0, 0)),
        out_shape=jax.ShapeDtypeStruct(
            (_NUM_ROW_BLOCKS, 1, _ROW_BLOCK), jnp.int32),
    )(flat, embeddings, inputs_sqr, emb_sqr)
    return out.reshape(_ROWS // _IDX_CHUNK, _IDX_CHUNK)


def _sc_gather_body(idx_hbm, table_hbm, out_hbm, idx_v, rows_v, sem):
    wid = lax.axis_index("s") * _SC_CORES + lax.axis_index("c")
    base = wid * _CHUNKS_PER_WORKER
    pltpu.sync_copy(idx_hbm.at[pl.ds(base, _CHUNKS_PER_WORKER)], idx_v)
    copies = []
    for j in range(_CHUNKS_PER_WORKER):
        copies.append(
            pltpu.async_copy(table_hbm.at[idx_v.at[j]], rows_v.at[j], sem))
    for c in copies:
        c.wait()
    pltpu.sync_copy(rows_v, out_hbm.at[pl.ds(base, _CHUNKS_PER_WORKER)])


@functools.cache
def _sc_gather():
    return pl.kernel(
        _sc_gather_body,
        mesh=plsc.VectorSubcoreMesh(core_axis_name="c", subcore_axis_name="s"),
        out_type=jax.ShapeDtypeStruct(
            (_ROWS // _IDX_CHUNK, _IDX_CHUNK, _EMBEDDING_DIM), jnp.float32),
        scratch_types=[
            pltpu.VMEM((_CHUNKS_PER_WORKER, _IDX_CHUNK), jnp.int32),
            pltpu.VMEM((_CHUNKS_PER_WORKER, _IDX_CHUNK, _EMBEDDING_DIM),
                       jnp.float32),
            pltpu.SemaphoreType.DMA,
        ],
        compiler_params=pltpu.CompilerParams(use_tc_tiling_on_sc=False),
    )


def _epilogue_body(z_ref, q_ref, out_ref, loss_ref):
    z = z_ref[...]
    q = q_ref[...]
    diff = q - z
    out_ref[...] = z + diff
    m = jnp.sum(diff * diff) * (1.0 / float(_ROWS * _EMBEDDING_DIM))
    loss_ref[0, 0] = m + _COMMITMENT_COST * m


def _epilogue(flat, q):
    return pl.pallas_call(
        _epilogue_body,
        out_shape=(
            jax.ShapeDtypeStruct((_ROWS, _EMBEDDING_DIM), jnp.float32),
            jax.ShapeDtypeStruct((1, 1), jnp.float32),
        ),
        out_specs=(
            pl.BlockSpec(memory_space=pltpu.VMEM),
            pl.BlockSpec(memory_space=pltpu.SMEM),
        ),
    )(flat, q)


def kernel(z_latents, embeddings):
    z_shape = z_latents.shape
    flat = z_latents.reshape(-1, _EMBEDDING_DIM)
    inputs_sqr = jnp.sum(flat**2, axis=1, keepdims=True)
    emb_sqr = jnp.sum(embeddings**2, axis=0, keepdims=True)
    idx = _argmin_indices(flat, embeddings, inputs_sqr, emb_sqr)
    table = embeddings.T
    q = _sc_gather()(idx, table).reshape(_ROWS, _EMBEDDING_DIM)
    out, loss = _epilogue(flat, q)
    return out.reshape(z_shape), loss[0, 0]


# speedup vs baseline: 3.8418x; 1.0084x over previous
import functools

import jax
import jax.numpy as jnp
from jax import lax
from jax.experimental import pallas as pl
from jax.experimental.pallas import tpu as pltpu
from jax.experimental.pallas import tpu_sc as plsc

_EMBEDDING_DIM = 32
_NUM_EMBEDDINGS = 8192
_COMMITMENT_COST = 0.25

_ROWS = 8192
_ROW_BLOCK = 1024
_NUM_ROW_BLOCKS = _ROWS // _ROW_BLOCK

_SC_CORES = 2
_SC_SUBCORES = 16
_SC_WORKERS = _SC_CORES * _SC_SUBCORES
_IDX_CHUNK = 128
_CHUNKS_PER_WORKER = _ROWS // (_SC_WORKERS * _IDX_CHUNK)


def _argmin_body(x_ref, e_ref, c_ref, s_ref, o_ref):
    x = x_ref[...]
    e = e_ref[...]
    dot = lax.dot_general(
        x, e, (((1,), (0,)), ((), ())),
        preferred_element_type=jnp.float32,
    )
    d = (c_ref[...] - 2.0 * dot) + s_ref[...]
    m = jnp.min(d, axis=1, keepdims=True)
    ji = lax.broadcasted_iota(jnp.int32, d.shape, 1)
    idx = jnp.min(jnp.where(d == m, ji, jnp.int32(2**31 - 1)), axis=1)
    o_ref[0, 0, :] = idx


def _argmin_indices(flat, embeddings, inputs_sqr, emb_sqr):
    out = pl.pallas_call(
        _argmin_body,
        grid=(_NUM_ROW_BLOCKS,),
        in_specs=[
            pl.BlockSpec((_ROW_BLOCK, _EMBEDDING_DIM), lambda i: (i, 0)),
            pl.BlockSpec((_EMBEDDING_DIM, _NUM_EMBEDDINGS), lambda i: (0, 0)),
            pl.BlockSpec((_ROW_BLOCK, 1), lambda i: (i, 0)),
            pl.BlockSpec((1, _NUM_EMBEDDINGS), lambda i: (0, 0)),
        ],
        out_specs=pl.BlockSpec((1, 1, _ROW_BLOCK), lambda i: (i, 0, 0)),
        out_shape=jax.ShapeDtypeStruct(
            (_NUM_ROW_BLOCKS, 1, _ROW_BLOCK), jnp.int32),
    )(flat, embeddings, inputs_sqr, emb_sqr)
    return out.reshape(_ROWS // _IDX_CHUNK, _IDX_CHUNK)


def _sc_gather_body(idx_hbm, table_hbm, out_hbm, idx_v, rows_v, sem):
    wid = lax.axis_index("s") * _SC_CORES + lax.axis_index("c")
    base = wid * _CHUNKS_PER_WORKER
    pltpu.sync_copy(idx_hbm.at[pl.ds(base, _CHUNKS_PER_WORKER)], idx_v)
    copies = []
    for j in range(_CHUNKS_PER_WORKER):
        copies.append(
            pltpu.async_copy(table_hbm.at[idx_v.at[j]], rows_v.at[j], sem))
    for c in copies:
        c.wait()
    pltpu.sync_copy(rows_v, out_hbm.at[pl.ds(base, _CHUNKS_PER_WORKER)])


@functools.cache
def _sc_gather():
    return pl.kernel(
        _sc_gather_body,
        mesh=plsc.VectorSubcoreMesh(core_axis_name="c", subcore_axis_name="s"),
        out_type=jax.ShapeDtypeStruct(
            (_ROWS // _IDX_CHUNK, _IDX_CHUNK, _EMBEDDING_DIM), jnp.float32),
        scratch_types=[
            pltpu.VMEM((_CHUNKS_PER_WORKER, _IDX_CHUNK), jnp.int32),
            pltpu.VMEM((_CHUNKS_PER_WORKER, _IDX_CHUNK, _EMBEDDING_DIM),
                       jnp.float32),
            pltpu.SemaphoreType.DMA,
        ],
        compiler_params=pltpu.CompilerParams(use_tc_tiling_on_sc=False),
    )


def _epilogue_body(z_ref, q_ref, out_ref, loss_ref):
    z = z_ref[...]
    q = q_ref[...]
    diff = q - z
    out_ref[...] = z + diff
    m = jnp.sum(diff * diff) * (1.0 / float(_ROWS * _EMBEDDING_DIM))
    loss_ref[0, 0] = m + _COMMITMENT_COST * m


def _epilogue(flat, q):
    return pl.pallas_call(
        _epilogue_body,
        out_shape=(
            jax.ShapeDtypeStruct((_ROWS, _EMBEDDING_DIM), jnp.float32),
            jax.ShapeDtypeStruct((1, 1), jnp.float32),
        ),
        out_specs=(
            pl.BlockSpec(memory_space=pltpu.VMEM),
            pl.BlockSpec(memory_space=pltpu.SMEM),
        ),
    )(flat, q)


def kernel(z_latents, embeddings):
    z_shape = z_latents.shape
    flat = z_latents.reshape(-1, _EMBEDDING_DIM)
    inputs_sqr = jnp.sum(flat**2, axis=1, keepdims=True)
    emb_sqr = jnp.sum(embeddings**2, axis=0, keepdims=True)
    idx = _argmin_indices(flat, embeddings, inputs_sqr, emb_sqr)
    table = embeddings.T
    q = _sc_gather()(idx, table).reshape(_ROWS, _EMBEDDING_DIM)
    out, loss = _epilogue(flat, q)
    return out.reshape(z_shape), loss[0, 0]


# speedup vs baseline: 3.8631x; 1.0056x over previous
import functools

import jax
import jax.numpy as jnp
from jax import lax
from jax.experimental import pallas as pl
from jax.experimental.pallas import tpu as pltpu
from jax.experimental.pallas import tpu_sc as plsc

_EMBEDDING_DIM = 32
_NUM_EMBEDDINGS = 8192
_COMMITMENT_COST = 0.25

_ROWS = 8192
_ROW_BLOCK = 1024
_NUM_ROW_BLOCKS = _ROWS // _ROW_BLOCK

_SC_CORES = 2
_SC_SUBCORES = 16
_SC_LANES = 16
_SC_WORKERS = _SC_CORES * _SC_SUBCORES
_IDX_CHUNK = 128
_CHUNKS_PER_WORKER = _ROWS // (_SC_WORKERS * _IDX_CHUNK)


def _argmin_body(x_ref, e_ref, c_ref, s_ref, o_ref):
    x = x_ref[...]
    e = e_ref[...]
    dot = lax.dot_general(
        x, e, (((1,), (0,)), ((), ())),
        preferred_element_type=jnp.float32,
    )
    d = (c_ref[...] - 2.0 * dot) + s_ref[...]
    m = jnp.min(d, axis=1, keepdims=True)
    ji = lax.broadcasted_iota(jnp.int32, d.shape, 1)
    idx = jnp.min(jnp.where(d == m, ji, jnp.int32(2**31 - 1)), axis=1)
    o_ref[0, 0, :] = idx


def _argmin_indices(flat, embeddings, inputs_sqr, emb_sqr):
    out = pl.pallas_call(
        _argmin_body,
        grid=(_NUM_ROW_BLOCKS,),
        in_specs=[
            pl.BlockSpec((_ROW_BLOCK, _EMBEDDING_DIM), lambda i: (i, 0)),
            pl.BlockSpec((_EMBEDDING_DIM, _NUM_EMBEDDINGS), lambda i: (0, 0)),
            pl.BlockSpec((_ROW_BLOCK, 1), lambda i: (i, 0)),
            pl.BlockSpec((1, _NUM_EMBEDDINGS), lambda i: (0, 0)),
        ],
        out_specs=pl.BlockSpec((1, 1, _ROW_BLOCK), lambda i: (i, 0, 0)),
        out_shape=jax.ShapeDtypeStruct(
            (_NUM_ROW_BLOCKS, 1, _ROW_BLOCK), jnp.int32),
    )(flat, embeddings, inputs_sqr, emb_sqr)
    return out.reshape(_ROWS // _IDX_CHUNK, _IDX_CHUNK)


def _sc_body(idx_hbm, table_hbm, z_hbm, out_hbm, part_hbm,
             idx_v, rows_v, z_v, out_v, acc_v, gsem, zsem):
    wid = lax.axis_index("s") * _SC_CORES + lax.axis_index("c")
    base = wid * _CHUNKS_PER_WORKER
    pltpu.sync_copy(idx_hbm.at[pl.ds(base, _CHUNKS_PER_WORKER)], idx_v)
    copies = [pltpu.async_copy(z_hbm.at[pl.ds(base, _CHUNKS_PER_WORKER)],
                               z_v, zsem)]
    for j in range(_CHUNKS_PER_WORKER):
        copies.append(
            pltpu.async_copy(table_hbm.at[idx_v.at[j]], rows_v.at[j], gsem))
    for c in copies:
        c.wait()

    def row_step(r, acc):
        for j in range(_CHUNKS_PER_WORKER):
            for h in range(_EMBEDDING_DIM // _SC_LANES):
                sl = pl.ds(h * _SC_LANES, _SC_LANES)
                q = rows_v[j, r, sl]
                z = z_v[j, r, sl]
                diff = q - z
                out_v[j, r, sl] = z + diff
                acc = acc + diff * diff
        return acc

    acc = lax.fori_loop(
        0, _IDX_CHUNK, row_step, jnp.zeros((_SC_LANES,), jnp.float32))
    acc_v[...] = acc
    pltpu.sync_copy(out_v, out_hbm.at[pl.ds(base, _CHUNKS_PER_WORKER)])
    pltpu.sync_copy(acc_v, part_hbm.at[wid])


@functools.cache
def _sc_gather_epilogue():
    return pl.kernel(
        _sc_body,
        mesh=plsc.VectorSubcoreMesh(core_axis_name="c", subcore_axis_name="s"),
        out_type=(
            jax.ShapeDtypeStruct(
                (_ROWS // _IDX_CHUNK, _IDX_CHUNK, _EMBEDDING_DIM),
                jnp.float32),
            jax.ShapeDtypeStruct((_SC_WORKERS, _SC_LANES), jnp.float32),
        ),
        scratch_types=[
            pltpu.VMEM((_CHUNKS_PER_WORKER, _IDX_CHUNK), jnp.int32),
            pltpu.VMEM((_CHUNKS_PER_WORKER, _IDX_CHUNK, _EMBEDDING_DIM),
                       jnp.float32),
            pltpu.VMEM((_CHUNKS_PER_WORKER, _IDX_CHUNK, _EMBEDDING_DIM),
                       jnp.float32),
            pltpu.VMEM((_CHUNKS_PER_WORKER, _IDX_CHUNK, _EMBEDDING_DIM),
                       jnp.float32),
            pltpu.VMEM((_SC_LANES,), jnp.float32),
            pltpu.SemaphoreType.DMA,
            pltpu.SemaphoreType.DMA,
        ],
        compiler_params=pltpu.CompilerParams(use_tc_tiling_on_sc=False),
    )


def kernel(z_latents, embeddings):
    z_shape = z_latents.shape
    flat = z_latents.reshape(-1, _EMBEDDING_DIM)
    inputs_sqr = jnp.sum(flat**2, axis=1, keepdims=True)
    emb_sqr = jnp.sum(embeddings**2, axis=0, keepdims=True)
    idx = _argmin_indices(flat, embeddings, inputs_sqr, emb_sqr)
    table = embeddings.T
    z3d = flat.reshape(_ROWS // _IDX_CHUNK, _IDX_CHUNK, _EMBEDDING_DIM)
    out3d, partials = _sc_gather_epilogue()(idx, table, z3d)
    m = jnp.sum(partials) * (1.0 / float(_ROWS * _EMBEDDING_DIM))
    loss = m + _COMMITMENT_COST * m
    return out3d.reshape(z_shape), loss
